# baseline (speedup 1.0000x reference)
import functools

import jax
import jax.numpy as jnp
from jax import lax
from jax.experimental import pallas as pl
from jax.experimental.pallas import tpu as pltpu
from jax.experimental.pallas import tpu_sc as plsc

BATCH = 16384
D = 64
HALF_OFFSET = 50000
NC = 2
NS = 16
L = 16
NW = NC * NS
BPW = BATCH // NW
CHUNK = 128
NCHUNK = BPW // CHUNK

_mesh = plsc.VectorSubcoreMesh(core_axis_name="c", subcore_axis_name="s")


@functools.partial(
    pl.kernel,
    mesh=_mesh,
    compiler_params=pltpu.CompilerParams(
        needs_layout_passes=False, use_tc_tiling_on_sc=False),
    out_type=jax.ShapeDtypeStruct((BATCH,), jnp.float32),
    scratch_types=[
        pltpu.VMEM((2 * BPW,), jnp.int32),
        pltpu.VMEM((NCHUNK, CHUNK), jnp.int32),
        pltpu.VMEM((NCHUNK, CHUNK), jnp.int32),
        pltpu.VMEM((BPW,), jnp.int32),
        pltpu.VMEM((BPW,), jnp.int32),
        pltpu.VMEM((2, CHUNK, 2 * D), jnp.float32),
        pltpu.VMEM((2, CHUNK, 2 * D), jnp.float32),
        pltpu.VMEM((80,), jnp.float32),
        pltpu.VMEM((BPW,), jnp.float32),
        pltpu.SemaphoreType.DMA,
        pltpu.SemaphoreType.DMA,
    ],
)
def _gmf_sc(x_hbm, t2_hbm, params_hbm, out_hbm,
            xv, uidx, vidx, upar, vpar, ublk, vblk, pv, outv, sem0, sem1):
    wid = lax.axis_index("s") * NC + lax.axis_index("c")
    base = wid * BPW

    pltpu.sync_copy(x_hbm.at[pl.ds(2 * base, 2 * BPW)], xv)
    pltpu.sync_copy(params_hbm, pv)

    lanes2 = lax.iota(jnp.int32, L) * 2
    for g in range(BPW // L):
        u = plsc.load_gather(xv, [lanes2 + (2 * L * g)])
        v = plsc.load_gather(xv, [lanes2 + (2 * L * g + 1)])
        r, c = g // (CHUNK // L), (g % (CHUNK // L)) * L
        uidx[r, pl.ds(c, L)] = u >> 1
        vidx[r, pl.ds(c, L)] = (v >> 1) + HALF_OFFSET
        upar[pl.ds(g * L, L)] = (u & 1) * D
        vpar[pl.ds(g * L, L)] = (v & 1) * D

    sems = (sem0, sem1)

    def fire(c):
        b = c % 2
        return [
            pltpu.async_copy(t2_hbm.at[uidx.at[c]], ublk.at[b], sems[b]),
            pltpu.async_copy(t2_hbm.at[vidx.at[c]], vblk.at[b], sems[b]),
        ]

    w0 = pv[pl.ds(0, L)]
    w1 = pv[pl.ds(16, L)]
    w2 = pv[pl.ds(32, L)]
    w3 = pv[pl.ds(48, L)]
    ws = (w0, w1, w2, w3)
    bias = pv[pl.ds(64, L)][0]
    lanes = lax.iota(jnp.int32, L)

    pending = fire(0)
    for c in range(NCHUNK):
        nxt = fire(c + 1) if c + 1 < NCHUNK else []
        for cp in pending:
            cp.wait()
        pending = nxt
        b = c % 2

        def group(g, carry, c=c, b=b):
            upv = upar[pl.ds(c * CHUNK + g * L, L)]
            vpv = vpar[pl.ds(c * CHUNK + g * L, L)]
            res = jnp.zeros((L,), jnp.float32)
            for j in range(L):
                i = g * L + j
                po_u = upv[j]
                po_v = vpv[j]
                acc = (ublk[b, i, pl.ds(po_u, L)]
                       * vblk[b, i, pl.ds(po_v, L)]) * w0
                for k in range(1, 4):
                    acc = acc + (ublk[b, i, pl.ds(po_u + k * L, L)]
                                 * vblk[b, i, pl.ds(po_v + k * L, L)]) * ws[k]
                res = jnp.where(lanes == j, jnp.sum(acc), res)
            outv[pl.ds(c * CHUNK + g * L, L)] = jnp.maximum(res + bias, 0.0)
            return carry

        lax.fori_loop(0, CHUNK // L, group, 0)

    pltpu.sync_copy(outv, out_hbm.at[pl.ds(base, BPW)])


def kernel(x, table, W, b):
    xflat = x.astype(jnp.int32).reshape(2 * BATCH)
    t2 = table.reshape(2 * HALF_OFFSET, 2 * D)
    params = jnp.concatenate(
        [W.reshape(D).astype(jnp.float32), b.astype(jnp.float32),
         jnp.zeros((15,), jnp.float32)])
    out = _gmf_sc(xflat, t2, params)
    return out.reshape(BATCH, 1)


# speedup vs baseline: 1.1391x; 1.1391x over previous
import functools

import jax
import jax.numpy as jnp
from jax import lax
from jax.experimental import pallas as pl
from jax.experimental.pallas import tpu as pltpu
from jax.experimental.pallas import tpu_sc as plsc

BATCH = 16384
D = 64
OFFSET = 100000
ROWS = 2 * OFFSET
NC = 2
NS = 16
L = 16
NW = NC * NS
BPW = BATCH // NW
CHUNK = 128
NCHUNK = BPW // CHUNK

TCOL = 2048

_mesh = plsc.VectorSubcoreMesh(core_axis_name="c", subcore_axis_name="s")


def _transpose_body(tt_ref, out_ref):
    out_ref[:, 0:D] = tt_ref[...].T


_transpose = pl.pallas_call(
    _transpose_body,
    grid=(pl.cdiv(ROWS, TCOL),),
    in_specs=[pl.BlockSpec((D, TCOL), lambda i: (0, i))],
    out_specs=pl.BlockSpec((TCOL, 2 * D), lambda i: (i, 0)),
    out_shape=jax.ShapeDtypeStruct((ROWS, 2 * D), jnp.float32),
)


@functools.partial(
    pl.kernel,
    mesh=_mesh,
    compiler_params=pltpu.CompilerParams(
        needs_layout_passes=False, use_tc_tiling_on_sc=False),
    out_type=jax.ShapeDtypeStruct((BATCH,), jnp.float32),
    scratch_types=[
        pltpu.VMEM((2 * BPW,), jnp.int32),
        pltpu.VMEM((NCHUNK, CHUNK), jnp.int32),
        pltpu.VMEM((NCHUNK, CHUNK), jnp.int32),
        pltpu.VMEM((2, CHUNK, 2 * D), jnp.float32),
        pltpu.VMEM((2, CHUNK, 2 * D), jnp.float32),
        pltpu.VMEM((80,), jnp.float32),
        pltpu.VMEM((BPW,), jnp.float32),
        pltpu.SemaphoreType.DMA,
        pltpu.SemaphoreType.DMA,
    ],
)
def _gmf_sc(x_hbm, t3_hbm, params_hbm, out_hbm,
            xv, uidx, vidx, ublk, vblk, pv, outv, sem0, sem1):
    wid = lax.axis_index("s") * NC + lax.axis_index("c")
    base = wid * BPW

    pltpu.sync_copy(x_hbm.at[pl.ds(2 * base, 2 * BPW)], xv)
    pltpu.sync_copy(params_hbm, pv)

    lanes2 = lax.iota(jnp.int32, L) * 2
    for g in range(BPW // L):
        u = plsc.load_gather(xv, [lanes2 + (2 * L * g)])
        v = plsc.load_gather(xv, [lanes2 + (2 * L * g + 1)])
        r, c = g // (CHUNK // L), (g % (CHUNK // L)) * L
        uidx[r, pl.ds(c, L)] = u
        vidx[r, pl.ds(c, L)] = v + OFFSET

    sems = (sem0, sem1)

    def fire(c):
        b = c % 2
        return [
            pltpu.async_copy(t3_hbm.at[uidx.at[c]], ublk.at[b], sems[b]),
            pltpu.async_copy(t3_hbm.at[vidx.at[c]], vblk.at[b], sems[b]),
        ]

    w0 = pv[pl.ds(0, L)]
    w1 = pv[pl.ds(16, L)]
    w2 = pv[pl.ds(32, L)]
    w3 = pv[pl.ds(48, L)]
    ws = (w0, w1, w2, w3)
    bias = pv[pl.ds(64, L)][0]
    lanes = lax.iota(jnp.int32, L)

    pending = fire(0)
    for c in range(NCHUNK):
        nxt = fire(c + 1) if c + 1 < NCHUNK else []
        for cp in pending:
            cp.wait()
        pending = nxt
        b = c % 2

        def group(g, carry, c=c, b=b):
            res = jnp.zeros((L,), jnp.float32)
            for j in range(L):
                i = g * L + j
                acc = (ublk[b, i, pl.ds(0, L)] * vblk[b, i, pl.ds(0, L)]) * w0
                for k in range(1, 4):
                    acc = acc + (ublk[b, i, pl.ds(k * L, L)]
                                 * vblk[b, i, pl.ds(k * L, L)]) * ws[k]
                res = jnp.where(lanes == j, jnp.sum(acc), res)
            outv[pl.ds(c * CHUNK + g * L, L)] = jnp.maximum(res + bias, 0.0)
            return carry

        lax.fori_loop(0, CHUNK // L, group, 0)

    pltpu.sync_copy(outv, out_hbm.at[pl.ds(base, BPW)])


def kernel(x, table, W, b):
    xflat = x.astype(jnp.int32).reshape(2 * BATCH)
    t3 = _transpose(table.T)
    params = jnp.concatenate(
        [W.reshape(D).astype(jnp.float32), b.astype(jnp.float32),
         jnp.zeros((15,), jnp.float32)])
    out = _gmf_sc(xflat, t3, params)
    return out.reshape(BATCH, 1)


# speedup vs baseline: 1.2898x; 1.1322x over previous
import functools

import jax
import jax.numpy as jnp
from jax import lax
from jax.experimental import pallas as pl
from jax.experimental.pallas import tpu as pltpu
from jax.experimental.pallas import tpu_sc as plsc

BATCH = 16384
D = 64
OFFSET = 100000
ROWS = 2 * OFFSET
NC = 2
NS = 16
L = 16
NW = NC * NS
BPW = BATCH // NW
CHUNK = 128
NCHUNK = BPW // CHUNK

SHIFT = OFFSET % 128
T2R = 782 * 128
M = 11
BLKR = 128 * M
RB = (OFFSET - SHIFT) // BLKR

_mesh = plsc.VectorSubcoreMesh(core_axis_name="c", subcore_axis_name="s")


def _transpose_body(l_ref, r_ref, out_ref):
    out_ref[:, 0:D] = l_ref[...].T
    out_ref[:, D:2 * D] = r_ref[...].T


_transpose = pl.pallas_call(
    _transpose_body,
    grid=(pl.cdiv(T2R, BLKR),),
    in_specs=[pl.BlockSpec((D, BLKR), lambda i: (0, i)),
              pl.BlockSpec((D, BLKR), lambda i: (0, i + RB))],
    out_specs=pl.BlockSpec((BLKR, 2 * D), lambda i: (i, 0)),
    out_shape=jax.ShapeDtypeStruct((T2R, 2 * D), jnp.float32),
)


@functools.partial(
    pl.kernel,
    mesh=_mesh,
    compiler_params=pltpu.CompilerParams(
        needs_layout_passes=False, use_tc_tiling_on_sc=False),
    out_type=jax.ShapeDtypeStruct((BATCH,), jnp.float32),
    scratch_types=[
        pltpu.VMEM((2 * BPW,), jnp.int32),
        pltpu.VMEM((NCHUNK, CHUNK), jnp.int32),
        pltpu.VMEM((NCHUNK, CHUNK), jnp.int32),
        pltpu.VMEM((2, CHUNK, 2 * D), jnp.float32),
        pltpu.VMEM((2, CHUNK, 2 * D), jnp.float32),
        pltpu.VMEM((80,), jnp.float32),
        pltpu.VMEM((BPW,), jnp.float32),
        pltpu.SemaphoreType.DMA,
        pltpu.SemaphoreType.DMA,
    ],
)
def _gmf_sc(x_hbm, t3_hbm, params_hbm, out_hbm,
            xv, uidx, vidx, ublk, vblk, pv, outv, sem0, sem1):
    wid = lax.axis_index("s") * NC + lax.axis_index("c")
    base = wid * BPW

    pltpu.sync_copy(x_hbm.at[pl.ds(2 * base, 2 * BPW)], xv)
    pltpu.sync_copy(params_hbm, pv)

    lanes2 = lax.iota(jnp.int32, L) * 2
    for g in range(BPW // L):
        u = plsc.load_gather(xv, [lanes2 + (2 * L * g)])
        v = plsc.load_gather(xv, [lanes2 + (2 * L * g + 1)])
        r, c = g // (CHUNK // L), (g % (CHUNK // L)) * L
        uidx[r, pl.ds(c, L)] = u
        vidx[r, pl.ds(c, L)] = v + SHIFT

    sems = (sem0, sem1)

    def fire(c):
        b = c % 2
        return [
            pltpu.async_copy(t3_hbm.at[uidx.at[c]], ublk.at[b], sems[b]),
            pltpu.async_copy(t3_hbm.at[vidx.at[c]], vblk.at[b], sems[b]),
        ]

    w0 = pv[pl.ds(0, L)]
    w1 = pv[pl.ds(16, L)]
    w2 = pv[pl.ds(32, L)]
    w3 = pv[pl.ds(48, L)]
    ws = (w0, w1, w2, w3)
    bias = pv[pl.ds(64, L)][0]
    lanes = lax.iota(jnp.int32, L)

    pending = fire(0)
    for c in range(NCHUNK):
        nxt = fire(c + 1) if c + 1 < NCHUNK else []
        for cp in pending:
            cp.wait()
        pending = nxt
        b = c % 2

        def group(g, carry, c=c, b=b):
            res = jnp.zeros((L,), jnp.float32)
            for j in range(L):
                i = g * L + j
                acc = (ublk[b, i, pl.ds(0, L)]
                       * vblk[b, i, pl.ds(D, L)]) * w0
                for k in range(1, 4):
                    acc = acc + (ublk[b, i, pl.ds(k * L, L)]
                                 * vblk[b, i, pl.ds(D + k * L, L)]) * ws[k]
                res = jnp.where(lanes == j, jnp.sum(acc), res)
            outv[pl.ds(c * CHUNK + g * L, L)] = jnp.maximum(res + bias, 0.0)
            return carry

        lax.fori_loop(0, CHUNK // L, group, 0)

    pltpu.sync_copy(outv, out_hbm.at[pl.ds(base, BPW)])


def kernel(x, table, W, b):
    xflat = x.astype(jnp.int32).reshape(2 * BATCH)
    tt = table.T
    t3 = _transpose(tt, tt)
    params = jnp.concatenate(
        [W.reshape(D).astype(jnp.float32), b.astype(jnp.float32),
         jnp.zeros((15,), jnp.float32)])
    out = _gmf_sc(xflat, t3, params)
    return out.reshape(BATCH, 1)


# speedup vs baseline: 1.9652x; 1.5236x over previous
import functools

import jax
import jax.numpy as jnp
from jax import lax
from jax.experimental import pallas as pl
from jax.experimental.pallas import tpu as pltpu
from jax.experimental.pallas import tpu_sc as plsc

BATCH = 16384
D = 64
OFFSET = 100000
ROWS = 2 * OFFSET
NC = 2
NS = 16
L = 16
NW = NC * NS
BPW = BATCH // NW
CHUNK = 128
NCHUNK = BPW // CHUNK

SHIFT = OFFSET % 128
T2R = 782 * 128
M = 71
BLKR = 128 * M
RB = (OFFSET - SHIFT) // BLKR

_mesh = plsc.VectorSubcoreMesh(core_axis_name="c", subcore_axis_name="s")


def _transpose_body(l_ref, r_ref, eye_ref, out_ref):
    del eye_ref
    x = jnp.concatenate([l_ref[...], r_ref[...]], axis=0)
    out_ref[...] = x.T


_transpose = pl.pallas_call(
    _transpose_body,
    grid=(pl.cdiv(T2R, BLKR),),
    in_specs=[pl.BlockSpec((D, BLKR), lambda i: (0, i)),
              pl.BlockSpec((D, BLKR), lambda i: (0, i + RB)),
              pl.BlockSpec((2 * D, 2 * D), lambda i: (0, 0))],
    out_specs=pl.BlockSpec((BLKR, 2 * D), lambda i: (i, 0)),
    out_shape=jax.ShapeDtypeStruct((T2R, 2 * D), jnp.float32),
)


@functools.partial(
    pl.kernel,
    mesh=_mesh,
    compiler_params=pltpu.CompilerParams(
        needs_layout_passes=False, use_tc_tiling_on_sc=False),
    out_type=jax.ShapeDtypeStruct((BATCH,), jnp.float32),
    scratch_types=[
        pltpu.VMEM((2 * BPW,), jnp.int32),
        pltpu.VMEM((NCHUNK, CHUNK), jnp.int32),
        pltpu.VMEM((NCHUNK, CHUNK), jnp.int32),
        pltpu.VMEM((2, CHUNK, 2 * D), jnp.float32),
        pltpu.VMEM((2, CHUNK, 2 * D), jnp.float32),
        pltpu.VMEM((80,), jnp.float32),
        pltpu.VMEM((BPW,), jnp.float32),
        pltpu.SemaphoreType.DMA,
        pltpu.SemaphoreType.DMA,
    ],
)
def _gmf_sc(x_hbm, t3_hbm, params_hbm, out_hbm,
            xv, uidx, vidx, ublk, vblk, pv, outv, sem0, sem1):
    wid = lax.axis_index("s") * NC + lax.axis_index("c")
    base = wid * BPW

    pltpu.sync_copy(x_hbm.at[pl.ds(2 * base, 2 * BPW)], xv)
    pltpu.sync_copy(params_hbm, pv)

    lanes2 = lax.iota(jnp.int32, L) * 2
    for g in range(BPW // L):
        u = plsc.load_gather(xv, [lanes2 + (2 * L * g)])
        v = plsc.load_gather(xv, [lanes2 + (2 * L * g + 1)])
        r, c = g // (CHUNK // L), (g % (CHUNK // L)) * L
        uidx[r, pl.ds(c, L)] = u
        vidx[r, pl.ds(c, L)] = v + SHIFT

    sems = (sem0, sem1)

    def fire(c):
        b = c % 2
        return [
            pltpu.async_copy(t3_hbm.at[uidx.at[c]], ublk.at[b], sems[b]),
            pltpu.async_copy(t3_hbm.at[vidx.at[c]], vblk.at[b], sems[b]),
        ]

    w0 = pv[pl.ds(0, L)]
    w1 = pv[pl.ds(16, L)]
    w2 = pv[pl.ds(32, L)]
    w3 = pv[pl.ds(48, L)]
    ws = (w0, w1, w2, w3)
    bias = pv[pl.ds(64, L)][0]
    lanes = lax.iota(jnp.int32, L)

    pending = fire(0)
    for c in range(NCHUNK):
        nxt = fire(c + 1) if c + 1 < NCHUNK else []
        for cp in pending:
            cp.wait()
        pending = nxt
        b = c % 2

        def group(g, carry, c=c, b=b):
            res = jnp.zeros((L,), jnp.float32)
            for j in range(L):
                i = g * L + j
                acc = (ublk[b, i, pl.ds(0, L)]
                       * vblk[b, i, pl.ds(D, L)]) * w0
                for k in range(1, 4):
                    acc = acc + (ublk[b, i, pl.ds(k * L, L)]
                                 * vblk[b, i, pl.ds(D + k * L, L)]) * ws[k]
                res = jnp.where(lanes == j, jnp.sum(acc), res)
            outv[pl.ds(c * CHUNK + g * L, L)] = jnp.maximum(res + bias, 0.0)
            return carry

        lax.fori_loop(0, CHUNK // L, group, 0)

    pltpu.sync_copy(outv, out_hbm.at[pl.ds(base, BPW)])


def kernel(x, table, W, b):
    xflat = x.astype(jnp.int32).reshape(2 * BATCH)
    tt = table.T
    t3 = _transpose(tt, tt, jnp.eye(2 * D, dtype=jnp.float32))
    params = jnp.concatenate(
        [W.reshape(D).astype(jnp.float32), b.astype(jnp.float32),
         jnp.zeros((15,), jnp.float32)])
    out = _gmf_sc(xflat, t3, params)
    return out.reshape(BATCH, 1)


# speedup vs baseline: 2.0000x; 1.0177x over previous
import functools

import jax
import jax.numpy as jnp
from jax import lax
from jax.experimental import pallas as pl
from jax.experimental.pallas import tpu as pltpu
from jax.experimental.pallas import tpu_sc as plsc

BATCH = 16384
D = 64
OFFSET = 100000
ROWS = 2 * OFFSET
NC = 2
NS = 16
L = 16
NW = NC * NS
BPW = BATCH // NW
CHUNK = 128
NCHUNK = BPW // CHUNK

SHIFT = OFFSET % 128
T2R = 782 * 128
M = 71
BLKR = 128 * M
RB = (OFFSET - SHIFT) // BLKR

_mesh = plsc.VectorSubcoreMesh(core_axis_name="c", subcore_axis_name="s")


def _transpose_body(l_ref, r_ref, out_ref):
    x = jnp.concatenate([l_ref[...], r_ref[...]], axis=0)
    out_ref[...] = x.T


_transpose = pl.pallas_call(
    _transpose_body,
    grid=(pl.cdiv(T2R, BLKR),),
    in_specs=[pl.BlockSpec((D, BLKR), lambda i: (0, i)),
              pl.BlockSpec((D, BLKR), lambda i: (0, i + RB))],
    out_specs=pl.BlockSpec((BLKR, 2 * D), lambda i: (i, 0)),
    out_shape=jax.ShapeDtypeStruct((T2R, 2 * D), jnp.float32),
)


@functools.partial(
    pl.kernel,
    mesh=_mesh,
    compiler_params=pltpu.CompilerParams(
        needs_layout_passes=False, use_tc_tiling_on_sc=False),
    out_type=jax.ShapeDtypeStruct((BATCH,), jnp.float32),
    scratch_types=[
        pltpu.VMEM((2 * BPW,), jnp.int32),
        pltpu.VMEM((NCHUNK, CHUNK), jnp.int32),
        pltpu.VMEM((NCHUNK, CHUNK), jnp.int32),
        pltpu.VMEM((3, CHUNK, 2 * D), jnp.float32),
        pltpu.VMEM((3, CHUNK, 2 * D), jnp.float32),
        pltpu.VMEM((80,), jnp.float32),
        pltpu.VMEM((BPW,), jnp.float32),
        pltpu.SemaphoreType.DMA,
        pltpu.SemaphoreType.DMA,
        pltpu.SemaphoreType.DMA,
    ],
)
def _gmf_sc(x_hbm, t3_hbm, params_hbm, out_hbm,
            xv, uidx, vidx, ublk, vblk, pv, outv, sem0, sem1, sem2):
    wid = lax.axis_index("s") * NC + lax.axis_index("c")
    base = wid * BPW

    pltpu.sync_copy(x_hbm.at[pl.ds(2 * base, 2 * BPW)], xv)
    pltpu.sync_copy(params_hbm, pv)

    lanes2 = lax.iota(jnp.int32, L) * 2
    for g in range(BPW // L):
        u = plsc.load_gather(xv, [lanes2 + (2 * L * g)])
        v = plsc.load_gather(xv, [lanes2 + (2 * L * g + 1)])
        r, c = g // (CHUNK // L), (g % (CHUNK // L)) * L
        uidx[r, pl.ds(c, L)] = u
        vidx[r, pl.ds(c, L)] = v + SHIFT

    sems = (sem0, sem1, sem2)

    def fire(c):
        b = c % 3
        return [
            pltpu.async_copy(t3_hbm.at[uidx.at[c]], ublk.at[b], sems[b]),
            pltpu.async_copy(t3_hbm.at[vidx.at[c]], vblk.at[b], sems[b]),
        ]

    w0 = pv[pl.ds(0, L)]
    w1 = pv[pl.ds(16, L)]
    w2 = pv[pl.ds(32, L)]
    w3 = pv[pl.ds(48, L)]
    ws = (w0, w1, w2, w3)
    bias = pv[pl.ds(64, L)][0]
    lanes = lax.iota(jnp.int32, L)

    inflight = {0: fire(0), 1: fire(1)}
    for c in range(NCHUNK):
        if c + 2 < NCHUNK:
            inflight[c + 2] = fire(c + 2)
        for cp in inflight.pop(c):
            cp.wait()
        b = c % 3

        def group(g, carry, c=c, b=b):
            res = jnp.zeros((L,), jnp.float32)
            for j in range(L):
                i = g * L + j
                acc = (ublk[b, i, pl.ds(0, L)]
                       * vblk[b, i, pl.ds(D, L)]) * w0
                for k in range(1, 4):
                    acc = acc + (ublk[b, i, pl.ds(k * L, L)]
                                 * vblk[b, i, pl.ds(D + k * L, L)]) * ws[k]
                res = jnp.where(lanes == j, jnp.sum(acc), res)
            outv[pl.ds(c * CHUNK + g * L, L)] = jnp.maximum(res + bias, 0.0)
            return carry

        lax.fori_loop(0, CHUNK // L, group, 0)

    pltpu.sync_copy(outv, out_hbm.at[pl.ds(base, BPW)])


def kernel(x, table, W, b):
    xflat = x.astype(jnp.int32).reshape(2 * BATCH)
    tt = table.T
    t3 = _transpose(tt, tt)
    params = jnp.concatenate(
        [W.reshape(D).astype(jnp.float32), b.astype(jnp.float32),
         jnp.zeros((15,), jnp.float32)])
    out = _gmf_sc(xflat, t3, params)
    return out.reshape(BATCH, 1)
